# baseline (speedup 1.0000x reference)
import functools

import jax
import jax.numpy as jnp
from jax import lax
from jax.experimental import pallas as pl
from jax.experimental.pallas import tpu as pltpu
from jax.experimental.pallas import tpu_sc as plsc

NC = 2
NS = 16
L = 16
NW = NC * NS

CHUNK = 128
N_PAD = 10112
ROWS_PER_TILE = N_PAD // NS


NBUF = 2
N_PHASES = 2


def _make_agg_kernel(n_chunks, d):
    mesh = plsc.VectorSubcoreMesh(core_axis_name="c", subcore_axis_name="s")
    assert n_chunks % (8 * N_PHASES) == 0
    pc = n_chunks // N_PHASES

    def body(x_hbm, src_hbm, dst_hbm, out_hbm, idx_src, idx_dst,
             rows0, rows1, agg_sh, sem0, sem1):
        rows = [rows0, rows1]
        sems = [sem0, sem1]
        cid = lax.axis_index("c")
        sid = lax.axis_index("s")
        wid = sid * NC + cid

        z = jnp.zeros((L,), jnp.float32)

        def zr(i, carry):
            def zc(j, c2):
                rows0[i, pl.ds(j * L, L)] = z
                return c2
            return lax.fori_loop(0, d // L, zc, carry)

        lax.fori_loop(0, CHUNK, zr, 0)

        base = sid * ROWS_PER_TILE
        nfull = ROWS_PER_TILE // CHUNK
        rem = ROWS_PER_TILE - nfull * CHUNK
        for k in range(nfull):
            pltpu.sync_copy(rows0, agg_sh.at[pl.ds(base + k * CHUNK, CHUNK)])
        if rem:
            pltpu.sync_copy(rows0.at[pl.ds(0, rem)],
                            agg_sh.at[pl.ds(base + nfull * CHUNK, rem)])

        plsc.subcore_barrier()

        for phase in range(N_PHASES):
            pltpu.sync_copy(src_hbm.at[wid].at[pl.ds(phase * pc, pc)],
                            idx_src)
            pltpu.sync_copy(dst_hbm.at[wid].at[pl.ds(phase * pc, pc)],
                            idx_dst)

            for b in range(NBUF):
                pltpu.async_copy(x_hbm.at[idx_src.at[b]], rows[b], sems[b])

            def outer(g, carry):
                for b in range(NBUF):
                    j = g * NBUF + b
                    pltpu.make_async_copy(x_hbm.at[pl.ds(0, CHUNK)], rows[b],
                                          sems[b]).wait()
                    pltpu.sync_copy(rows[b], agg_sh.at[idx_dst.at[j]],
                                    add=True)

                    @pl.when(j + NBUF < pc)
                    def _refill(b=b, j=j):
                        pltpu.async_copy(x_hbm.at[idx_src.at[j + NBUF]],
                                         rows[b], sems[b])
                return carry

            lax.fori_loop(0, pc // NBUF, outer, 0)

        plsc.subcore_barrier()

        pltpu.sync_copy(agg_sh.at[pl.ds(base, ROWS_PER_TILE)],
                        out_hbm.at[cid].at[pl.ds(base, ROWS_PER_TILE)])

    return pl.kernel(
        body,
        out_type=jax.ShapeDtypeStruct((NC, N_PAD, d), jnp.float32),
        mesh=mesh,
        scratch_types=[
            pltpu.VMEM((pc, CHUNK), jnp.int32),
            pltpu.VMEM((pc, CHUNK), jnp.int32),
            pltpu.VMEM((CHUNK, d), jnp.float32),
            pltpu.VMEM((CHUNK, d), jnp.float32),
            pltpu.VMEM_SHARED((N_PAD, d), jnp.float32),
            pltpu.SemaphoreType.DMA,
            pltpu.SemaphoreType.DMA,
        ],
    )


def _ln(h, g, b, eps=1e-5):
    mean = jnp.mean(h, axis=-1, keepdims=True)
    c = h - mean
    var = jnp.mean(c * c, axis=-1, keepdims=True)
    return c * lax.rsqrt(var + eps) * g + b


def _mlp_body(scale_ref, x_ref, a0_ref, a1_ref, W1_ref, b1_ref, g1_ref,
              be1_ref, W2_ref, b2_ref, g2_ref, be2_ref, W3_ref, b3_ref,
              gb_ref, bb_ref, y_ref):
    x = x_ref[...]
    out = scale_ref[0] * x + a0_ref[...] + a1_ref[...]
    h = jnp.dot(out, W1_ref[...], preferred_element_type=jnp.float32)
    h = _ln(h + b1_ref[...], g1_ref[...], be1_ref[...])
    h = jnp.maximum(h, 0.0)
    h = jnp.dot(h, W2_ref[...], preferred_element_type=jnp.float32)
    h = _ln(h + b2_ref[...], g2_ref[...], be2_ref[...])
    h = jnp.maximum(h, 0.0)
    h = jnp.dot(h, W3_ref[...], preferred_element_type=jnp.float32)
    y = _ln(x + h + b3_ref[...], gb_ref[...], bb_ref[...])
    y_ref[...] = jnp.maximum(y, 0.0)


def _mlp_call(scale, x, a0, a1, W1, b1, g1, be1, W2, b2, g2, be2, W3, b3,
              gb, bb, row_blk):
    n, d = x.shape
    h = W1.shape[1]
    grid = (n // row_blk,)
    rows = pl.BlockSpec((row_blk, d), lambda i: (i, 0))
    full = lambda r, c: pl.BlockSpec((r, c), lambda i: (0, 0))
    return pl.pallas_call(
        _mlp_body,
        grid=grid,
        in_specs=[
            pl.BlockSpec(memory_space=pltpu.SMEM),
            rows, rows, rows,
            full(d, h), full(1, h), full(1, h), full(1, h),
            full(h, h), full(1, h), full(1, h), full(1, h),
            full(h, d), full(1, d), full(1, d), full(1, d),
        ],
        out_specs=rows,
        out_shape=jax.ShapeDtypeStruct((n, d), jnp.float32),
    )(scale, x, a0, a1, W1, b1, g1, be1, W2, b2, g2, be2, W3, b3, gb, bb)


def kernel(x, edge_index, W1, b1, ln1_g, ln1_b, W2, b2, ln2_g, ln2_b, W3, b3,
           eps, blk_g, blk_b):
    n, d = x.shape
    e = edge_index.shape[1]
    n_chunks = -(-e // (NW * CHUNK))
    align = 8 * N_PHASES
    n_chunks = -(-n_chunks // align) * align
    e_pad = NW * n_chunks * CHUNK
    src = edge_index[0]
    dst = edge_index[1]
    if e_pad > e:
        pad = e_pad - e
        src = jnp.concatenate([src, jnp.zeros((pad,), jnp.int32)])
        dst = jnp.concatenate([dst, jnp.full((pad,), n, jnp.int32)])
    src_t = src.reshape(NW, n_chunks, CHUNK)
    dst_t = dst.reshape(NW, n_chunks, CHUNK)

    partials = _make_agg_kernel(n_chunks, d)(x, src_t, dst_t)
    a0 = partials[0, :n]
    a1 = partials[1, :n]

    scale = jnp.reshape(2.0 + eps, (1,))
    r2 = lambda v: v.reshape(1, -1)
    return _mlp_call(scale, x, a0, a1, W1, r2(b1), r2(ln1_g), r2(ln1_b),
                     W2, r2(b2), r2(ln2_g), r2(ln2_b), W3, r2(b3),
                     r2(blk_g), r2(blk_b), row_blk=2000)


# speedup vs baseline: 1.8368x; 1.8368x over previous
import functools

import jax
import jax.numpy as jnp
from jax import lax
from jax.experimental import pallas as pl
from jax.experimental.pallas import tpu as pltpu
from jax.experimental.pallas import tpu_sc as plsc

NC = 2
NS = 16
L = 16
NW = NC * NS

CHUNK = 128
N_PAD = 10112
ROWS_PER_TILE = N_PAD // NS


C0_FRAC = 0.65


def _make_agg_kernel(c0, c1, cmax, d):
    mesh = plsc.VectorSubcoreMesh(core_axis_name="c", subcore_axis_name="s")

    def body(x_hbm, src_hbm, dst_hbm, out_hbm, idx_src, idx_dst, rows,
             agg_sh, sem):
        cid = lax.axis_index("c")
        sid = lax.axis_index("s")
        wid = cid * NS + sid
        nch = jnp.where(cid == 0, c0, c1)

        z = jnp.zeros((L,), jnp.float32)

        def zr(i, carry):
            def zc(j, c2):
                rows[i, pl.ds(j * L, L)] = z
                return c2
            return lax.fori_loop(0, d // L, zc, carry)

        lax.fori_loop(0, CHUNK, zr, 0)

        base = sid * ROWS_PER_TILE
        nfull = ROWS_PER_TILE // CHUNK
        rem = ROWS_PER_TILE - nfull * CHUNK
        for k in range(nfull):
            pltpu.sync_copy(rows, agg_sh.at[pl.ds(base + k * CHUNK, CHUNK)])
        if rem:
            pltpu.sync_copy(rows.at[pl.ds(0, rem)],
                            agg_sh.at[pl.ds(base + nfull * CHUNK, rem)])

        pltpu.sync_copy(src_hbm.at[wid], idx_src)
        pltpu.sync_copy(dst_hbm.at[wid], idx_dst)

        plsc.subcore_barrier()

        def step(j, carry):
            pltpu.async_copy(x_hbm.at[idx_src.at[j]], rows, sem).wait()
            pltpu.sync_copy(rows, agg_sh.at[idx_dst.at[j]], add=True)
            return carry

        lax.fori_loop(0, nch, step, 0)

        plsc.subcore_barrier()

        pltpu.sync_copy(agg_sh.at[pl.ds(base, ROWS_PER_TILE)],
                        out_hbm.at[cid].at[pl.ds(base, ROWS_PER_TILE)])

    return pl.kernel(
        body,
        out_type=jax.ShapeDtypeStruct((NC, N_PAD, d), jnp.float32),
        mesh=mesh,
        scratch_types=[
            pltpu.VMEM((cmax, CHUNK), jnp.int32),
            pltpu.VMEM((cmax, CHUNK), jnp.int32),
            pltpu.VMEM((CHUNK, d), jnp.float32),
            pltpu.VMEM_SHARED((N_PAD, d), jnp.float32),
            pltpu.SemaphoreType.DMA,
        ],
    )


def _ln(h, g, b, eps=1e-5):
    mean = jnp.mean(h, axis=-1, keepdims=True)
    c = h - mean
    var = jnp.mean(c * c, axis=-1, keepdims=True)
    return c * lax.rsqrt(var + eps) * g + b


def _mlp_body(scale_ref, x_ref, a0_ref, a1_ref, W1_ref, b1_ref, g1_ref,
              be1_ref, W2_ref, b2_ref, g2_ref, be2_ref, W3_ref, b3_ref,
              gb_ref, bb_ref, y_ref):
    x = x_ref[...]
    out = scale_ref[0] * x + a0_ref[...] + a1_ref[...]
    h = jnp.dot(out, W1_ref[...], preferred_element_type=jnp.float32)
    h = _ln(h + b1_ref[...], g1_ref[...], be1_ref[...])
    h = jnp.maximum(h, 0.0)
    h = jnp.dot(h, W2_ref[...], preferred_element_type=jnp.float32)
    h = _ln(h + b2_ref[...], g2_ref[...], be2_ref[...])
    h = jnp.maximum(h, 0.0)
    h = jnp.dot(h, W3_ref[...], preferred_element_type=jnp.float32)
    y = _ln(x + h + b3_ref[...], gb_ref[...], bb_ref[...])
    y_ref[...] = jnp.maximum(y, 0.0)


def _mlp_call(scale, x, a0, a1, W1, b1, g1, be1, W2, b2, g2, be2, W3, b3,
              gb, bb, row_blk):
    n, d = x.shape
    h = W1.shape[1]
    grid = (n // row_blk,)
    rows = pl.BlockSpec((row_blk, d), lambda i: (i, 0))
    full = lambda r, c: pl.BlockSpec((r, c), lambda i: (0, 0))
    return pl.pallas_call(
        _mlp_body,
        grid=grid,
        in_specs=[
            pl.BlockSpec(memory_space=pltpu.SMEM),
            rows, rows, rows,
            full(d, h), full(1, h), full(1, h), full(1, h),
            full(h, h), full(1, h), full(1, h), full(1, h),
            full(h, d), full(1, d), full(1, d), full(1, d),
        ],
        out_specs=rows,
        out_shape=jax.ShapeDtypeStruct((n, d), jnp.float32),
    )(scale, x, a0, a1, W1, b1, g1, be1, W2, b2, g2, be2, W3, b3, gb, bb)


def kernel(x, edge_index, W1, b1, ln1_g, ln1_b, W2, b2, ln2_g, ln2_b, W3, b3,
           eps, blk_g, blk_b):
    n, d = x.shape
    e = edge_index.shape[1]
    tot = -(-e // (NS * CHUNK))
    c0 = int(round(tot * C0_FRAC))
    c1 = tot - c0
    cmax = max(c0, c1)
    e_pad = NS * tot * CHUNK
    src = edge_index[0]
    dst = edge_index[1]
    if e_pad > e:
        pad = e_pad - e
        src = jnp.concatenate([src, jnp.zeros((pad,), jnp.int32)])
        dst = jnp.concatenate([dst, jnp.full((pad,), n, jnp.int32)])

    def split(v):
        ec0 = NS * c0 * CHUNK
        p0 = v[:ec0].reshape(NS, c0, CHUNK)
        p1 = v[ec0:].reshape(NS, c1, CHUNK)
        p0 = jnp.pad(p0, ((0, 0), (0, cmax - c0), (0, 0)))
        p1 = jnp.pad(p1, ((0, 0), (0, cmax - c1), (0, 0)))
        return jnp.concatenate([p0, p1], axis=0)

    partials = _make_agg_kernel(c0, c1, cmax, d)(x, split(src), split(dst))
    a0 = partials[0, :n]
    a1 = partials[1, :n]

    scale = jnp.reshape(2.0 + eps, (1,))
    r2 = lambda v: v.reshape(1, -1)
    return _mlp_call(scale, x, a0, a1, W1, r2(b1), r2(ln1_g), r2(ln1_b),
                     W2, r2(b2), r2(ln2_g), r2(ln2_b), W3, r2(b3),
                     r2(blk_g), r2(blk_b), row_blk=2000)


# speedup vs baseline: 1.9569x; 1.0654x over previous
import functools

import jax
import jax.numpy as jnp
from jax import lax
from jax.experimental import pallas as pl
from jax.experimental.pallas import tpu as pltpu
from jax.experimental.pallas import tpu_sc as plsc

NC = 2
NS = 16
L = 16
NW = NC * NS

CHUNK = 128
N_PAD = 10112
ROWS_PER_TILE = N_PAD // NS


C0_FRAC = 0.61


def _make_agg_kernel(c0, c1, cmax, d):
    mesh = plsc.VectorSubcoreMesh(core_axis_name="c", subcore_axis_name="s")

    def body(x_hbm, src_hbm, dst_hbm, out_hbm, idx_src, idx_dst, rows,
             agg_sh, sem):
        cid = lax.axis_index("c")
        sid = lax.axis_index("s")
        wid = cid * NS + sid
        nch = jnp.where(cid == 0, c0, c1)

        z = jnp.zeros((L,), jnp.float32)

        def zr(i, carry):
            def zc(j, c2):
                rows[i, pl.ds(j * L, L)] = z
                return c2
            return lax.fori_loop(0, d // L, zc, carry)

        lax.fori_loop(0, CHUNK, zr, 0)

        base = sid * ROWS_PER_TILE
        nfull = ROWS_PER_TILE // CHUNK
        rem = ROWS_PER_TILE - nfull * CHUNK
        for k in range(nfull):
            pltpu.sync_copy(rows, agg_sh.at[pl.ds(base + k * CHUNK, CHUNK)])
        if rem:
            pltpu.sync_copy(rows.at[pl.ds(0, rem)],
                            agg_sh.at[pl.ds(base + nfull * CHUNK, rem)])

        pltpu.sync_copy(src_hbm.at[wid], idx_src)
        pltpu.sync_copy(dst_hbm.at[wid], idx_dst)

        plsc.subcore_barrier()

        def step(j, carry):
            pltpu.async_copy(x_hbm.at[idx_src.at[j]], rows, sem).wait()
            pltpu.sync_copy(rows, agg_sh.at[idx_dst.at[j]], add=True)
            return carry

        lax.fori_loop(0, nch, step, 0)

        plsc.subcore_barrier()

        pltpu.sync_copy(agg_sh.at[pl.ds(base, ROWS_PER_TILE)],
                        out_hbm.at[cid].at[pl.ds(base, ROWS_PER_TILE)])

    return pl.kernel(
        body,
        out_type=jax.ShapeDtypeStruct((NC, N_PAD, d), jnp.float32),
        mesh=mesh,
        scratch_types=[
            pltpu.VMEM((cmax, CHUNK), jnp.int32),
            pltpu.VMEM((cmax, CHUNK), jnp.int32),
            pltpu.VMEM((CHUNK, d), jnp.float32),
            pltpu.VMEM_SHARED((N_PAD, d), jnp.float32),
            pltpu.SemaphoreType.DMA,
        ],
    )


def _ln(h, g, b, eps=1e-5):
    mean = jnp.mean(h, axis=-1, keepdims=True)
    c = h - mean
    var = jnp.mean(c * c, axis=-1, keepdims=True)
    return c * lax.rsqrt(var + eps) * g + b


def _mlp_body(scale_ref, x_ref, a0_ref, a1_ref, W1_ref, b1_ref, g1_ref,
              be1_ref, W2_ref, b2_ref, g2_ref, be2_ref, W3_ref, b3_ref,
              gb_ref, bb_ref, y_ref):
    x = x_ref[...]
    out = scale_ref[0] * x + a0_ref[...] + a1_ref[...]
    h = jnp.dot(out, W1_ref[...], preferred_element_type=jnp.float32)
    h = _ln(h + b1_ref[...], g1_ref[...], be1_ref[...])
    h = jnp.maximum(h, 0.0)
    h = jnp.dot(h, W2_ref[...], preferred_element_type=jnp.float32)
    h = _ln(h + b2_ref[...], g2_ref[...], be2_ref[...])
    h = jnp.maximum(h, 0.0)
    h = jnp.dot(h, W3_ref[...], preferred_element_type=jnp.float32)
    y = _ln(x + h + b3_ref[...], gb_ref[...], bb_ref[...])
    y_ref[...] = jnp.maximum(y, 0.0)


def _mlp_call(scale, x, a0, a1, W1, b1, g1, be1, W2, b2, g2, be2, W3, b3,
              gb, bb, row_blk):
    n, d = x.shape
    h = W1.shape[1]
    grid = (n // row_blk,)
    rows = pl.BlockSpec((row_blk, d), lambda i: (i, 0))
    full = lambda r, c: pl.BlockSpec((r, c), lambda i: (0, 0))
    return pl.pallas_call(
        _mlp_body,
        grid=grid,
        in_specs=[
            pl.BlockSpec(memory_space=pltpu.SMEM),
            rows, rows, rows,
            full(d, h), full(1, h), full(1, h), full(1, h),
            full(h, h), full(1, h), full(1, h), full(1, h),
            full(h, d), full(1, d), full(1, d), full(1, d),
        ],
        out_specs=rows,
        out_shape=jax.ShapeDtypeStruct((n, d), jnp.float32),
    )(scale, x, a0, a1, W1, b1, g1, be1, W2, b2, g2, be2, W3, b3, gb, bb)


def kernel(x, edge_index, W1, b1, ln1_g, ln1_b, W2, b2, ln2_g, ln2_b, W3, b3,
           eps, blk_g, blk_b):
    n, d = x.shape
    e = edge_index.shape[1]
    tot = -(-e // (NS * CHUNK))
    c0 = int(round(tot * C0_FRAC))
    c1 = tot - c0
    cmax = max(c0, c1)
    e_pad = NS * tot * CHUNK
    src = edge_index[0]
    dst = edge_index[1]
    if e_pad > e:
        pad = e_pad - e
        src = jnp.concatenate([src, jnp.zeros((pad,), jnp.int32)])
        dst = jnp.concatenate([dst, jnp.full((pad,), n, jnp.int32)])

    def split(v):
        ec0 = NS * c0 * CHUNK
        p0 = v[:ec0].reshape(NS, c0, CHUNK)
        p1 = v[ec0:].reshape(NS, c1, CHUNK)
        p0 = jnp.pad(p0, ((0, 0), (0, cmax - c0), (0, 0)))
        p1 = jnp.pad(p1, ((0, 0), (0, cmax - c1), (0, 0)))
        return jnp.concatenate([p0, p1], axis=0)

    partials = _make_agg_kernel(c0, c1, cmax, d)(x, split(src), split(dst))
    a0 = partials[0, :n]
    a1 = partials[1, :n]

    scale = jnp.reshape(2.0 + eps, (1,))
    r2 = lambda v: v.reshape(1, -1)
    return _mlp_call(scale, x, a0, a1, W1, r2(b1), r2(ln1_g), r2(ln1_b),
                     W2, r2(b2), r2(ln2_g), r2(ln2_b), W3, r2(b3),
                     r2(blk_g), r2(blk_b), row_blk=2000)
